# Initial kernel scaffold; baseline (speedup 1.0000x reference)
import jax
import jax.numpy as jnp
from jax.experimental import pallas as pl


def kernel(ch1, ch2, CP_locs, CP_idx):
    raise NotImplementedError("write your pallas kernel here")


# speedup vs baseline: 5.4506x; 5.4506x over previous
import functools

import jax
import jax.numpy as jnp
from jax import lax
from jax.experimental import pallas as pl
from jax.experimental.pallas import tpu as pltpu
from jax.experimental.pallas import tpu_sc as plsc

H = 2048
W = 2048
T = 640
LPC = 2 * T
GROUPS_A = T // 16
VECS_B = LPC // 16
JROWS = T // 128
NW = 32
L = 16
ROWW = 16
VROWS = H * W * 2 // ROWW


def _lanes():
  return lax.iota(jnp.int32, L)


def _splat(x):
  return jnp.broadcast_to(jnp.asarray(x, jnp.int32), (L,))


def _sc_body(idx_hbm, ch1_hbm, ch2_hbm, table_hbm, out_hbm,
             idxraw0, idxraw1, c1b0, c1b1, c2b0, c2b1,
             ib0, ib1, gb0, gb1, accv,
             sin0, sin1, sg0, sg1, nchunks):
  wid = lax.axis_index("s") * 2 + lax.axis_index("c")
  nt = (nchunks - wid + NW - 1) // NW
  lanes = _lanes()
  idxraws = (idxraw0, idxraw1)
  c1bs = (c1b0, c1b1)
  c2bs = (c2b0, c2b1)
  ibs = (ib0, ib1)
  gbs = (gb0, gb1)
  sins = (sin0, sin1)
  sgs = (sg0, sg1)

  def chunk_off(t):
    return (wid + NW * t) * LPC

  def s_in(t, b):
    off = chunk_off(t)
    pltpu.async_copy(idx_hbm.at[pl.ds(off, LPC)], idxraws[b], sins[b])
    pltpu.async_copy(ch1_hbm.at[pl.ds(off, LPC)], c1bs[b], sins[b])
    pltpu.async_copy(ch2_hbm.at[pl.ds(off, LPC)], c2bs[b], sins[b])

  def s_idx(t, b):
    off = chunk_off(t)
    pltpu.make_async_copy(idx_hbm.at[pl.ds(off, LPC)], idxraws[b],
                          sins[b]).wait()
    pltpu.make_async_copy(ch1_hbm.at[pl.ds(off, LPC)], c1bs[b],
                          sins[b]).wait()
    pltpu.make_async_copy(ch2_hbm.at[pl.ds(off, LPC)], c2bs[b],
                          sins[b]).wait()

    def build(u, _):
      posr = u * 32 + lanes * 2
      r = plsc.load_gather(idxraws[b], [posr])
      c = plsc.load_gather(idxraws[b], [posr + 1])
      base = r * W + c
      btop = base - W
      qt = lax.shift_right_logical(btop, 3)
      qb = lax.shift_right_logical(base, 3)
      q = _splat(u >> 3)
      o = _splat((u & 7) << 4) + lanes
      plsc.store_scatter(ibs[b], [_splat(0), q, o], qt)
      plsc.store_scatter(ibs[b], [_splat(1), q, o], qt + 1)
      plsc.store_scatter(ibs[b], [_splat(2), q, o], qb)
      plsc.store_scatter(ibs[b], [_splat(3), q, o], qb + 1)
      plsc.store_scatter(ibs[b], [_splat(4), q, o], base & 7)
      return _

    lax.fori_loop(0, GROUPS_A, build, None)
    for k in range(4):
      for j in range(JROWS):
        pltpu.async_copy(table_hbm.at[ibs[b].at[k, j]], gbs[b].at[k, j],
                         sgs[b])

  def s_cmp(t, b):
    for k in range(4):
      for j in range(JROWS):
        pltpu.make_async_copy(table_hbm.at[ibs[b].at[k, j]],
                              gbs[b].at[k, j], sgs[b]).wait()

    def compute(v, acc):
      jv = _splat(v >> 4)
      rowp = _splat((v & 15) << 3) + (lanes >> 1)
      col = lanes & 1
      a8 = plsc.load_gather(ibs[b], [_splat(4), jv, rowp])
      e01 = a8 * 2 + col
      stra = (a8 == 7).astype(jnp.int32)
      e23 = (e01 + 2) & 15
      k0 = stra
      k2 = stra + 2
      p0 = plsc.load_gather(gbs[b], [_splat(0), jv, rowp, e01])
      p3 = plsc.load_gather(gbs[b], [k0, jv, rowp, e23])
      p1 = plsc.load_gather(gbs[b], [_splat(2), jv, rowp, e01])
      p2 = plsc.load_gather(gbs[b], [k2, jv, rowp, e23])
      c1v = c1bs[b][pl.ds(v * 16, 16)]
      c2v = c2bs[b][pl.ds(v * 16, 16)]
      tt = c2v - p1
      ca = 1.5 * (p1 - p2) + 0.5 * (p3 - p0)
      cb = p0 - 2.5 * p1 + 2.0 * p2 - 0.5 * p3
      cc = 0.5 * (p2 - p0)
      mapped = ((ca * tt + cb) * tt + cc) * tt + p1
      d = c1v - mapped
      return acc + d * d

    local = lax.fori_loop(0, VECS_B, compute,
                          jnp.zeros((L,), jnp.float32))
    accv[...] = accv[...] + local

  accv[...] = jnp.zeros((L,), jnp.float32)

  s_in(0, 0)

  @pl.when(nt > 1)
  def _():
    s_in(1, 1)

  s_idx(0, 0)

  def step(kk, _):
    for b in (0, 1):
      t = 2 * kk + b

      @pl.when(t + 1 < nt)
      def _():
        s_idx(t + 1, 1 - b)

      @pl.when(t < nt)
      def _():
        s_cmp(t, b)

      @pl.when(t + 2 < nt)
      def _():
        s_in(t + 2, b)
    return _

  kmax = (nchunks // NW + 2) // 2 + 1
  lax.fori_loop(0, kmax, step, None)

  pltpu.sync_copy(accv, out_hbm.at[wid])


def _make_sc_call(n_points):
  nchunks = n_points // T
  mesh = plsc.VectorSubcoreMesh(core_axis_name="c", subcore_axis_name="s")
  body = functools.partial(_sc_body, nchunks=nchunks)
  return pl.kernel(
      body,
      out_type=jax.ShapeDtypeStruct((NW, L), jnp.float32),
      mesh=mesh,
      scratch_types=[
          pltpu.VMEM((LPC,), jnp.int32),
          pltpu.VMEM((LPC,), jnp.int32),
          pltpu.VMEM((LPC,), jnp.float32),
          pltpu.VMEM((LPC,), jnp.float32),
          pltpu.VMEM((LPC,), jnp.float32),
          pltpu.VMEM((LPC,), jnp.float32),
          pltpu.VMEM((5, JROWS, 128), jnp.int32),
          pltpu.VMEM((5, JROWS, 128), jnp.int32),
          pltpu.VMEM((4, JROWS, 128, ROWW), jnp.float32),
          pltpu.VMEM((4, JROWS, 128, ROWW), jnp.float32),
          pltpu.VMEM((L,), jnp.float32),
          pltpu.SemaphoreType.DMA,
          pltpu.SemaphoreType.DMA,
          pltpu.SemaphoreType.DMA,
          pltpu.SemaphoreType.DMA,
      ],
      compiler_params=pltpu.CompilerParams(
          use_tc_tiling_on_sc=False,
          needs_layout_passes=False,
      ),
  )


@jax.jit
def kernel(ch1, ch2, CP_locs, CP_idx):
  n = ch1.shape[0]
  table = CP_locs.reshape(VROWS, ROWW)
  call = _make_sc_call(n)
  partials = call(CP_idx.reshape(-1), ch1.reshape(-1), ch2.reshape(-1),
                  table)
  return 0.5 * jnp.sum(partials) / n


# speedup vs baseline: 116.3809x; 21.3519x over previous
import functools

import jax
import jax.numpy as jnp
from jax import lax
from jax.experimental import pallas as pl
from jax.experimental.pallas import tpu as pltpu
from jax.experimental.pallas import tpu_sc as plsc

H = 2048
W = 2048
T = 640
LPC = 2 * T
GROUPS_A = T // 16
VECS_B = LPC // 16
JROWS = T // 128
NW = 32
L = 16
ROWW = 16
VROWS = H * W * 2 // ROWW
SLAB = 2 * W
HPW = H // NW


def _lanes():
  return lax.iota(jnp.int32, L)


def _splat(x):
  return jnp.broadcast_to(jnp.asarray(x, jnp.int32), (L,))


def _relayout_body(x_hbm, o_hbm, in0, in1, ou0, ou1,
                   sin0, sin1, sou0, sou1):
  wid = lax.axis_index("s") * 2 + lax.axis_index("c")
  lanes = _lanes()
  lh = (lanes >> 1) + (lanes & 1) * 128
  ins = (in0, in1)
  ous = (ou0, ou1)
  sins = (sin0, sin1)
  sous = (sou0, sou1)

  def off(s):
    return (wid * HPW + s) * SLAB

  def s_in(s, b):
    pltpu.async_copy(x_hbm.at[pl.ds(off(s), SLAB)], ins[b], sins[b])

  def do(s, b):
    pltpu.make_async_copy(x_hbm.at[pl.ds(off(s), SLAB)], ins[b],
                          sins[b]).wait()

    @pl.when(s >= 2)
    def _():
      pltpu.make_async_copy(ous[b], o_hbm.at[pl.ds(off(s - 2), SLAB)],
                            sous[b]).wait()

    def shuf(q, _):
      idx = _splat((q >> 4) * 256 + (q & 15) * 8) + lh
      ous[b][pl.ds(q * 16, 16)] = plsc.load_gather(ins[b], [idx])
      return _

    lax.fori_loop(0, SLAB // 16, shuf, None)
    pltpu.async_copy(ous[b], o_hbm.at[pl.ds(off(s), SLAB)], sous[b])

  s_in(0, 0)
  s_in(1, 1)

  def step(kk, _):
    for b in (0, 1):
      s = 2 * kk + b

      @pl.when(s + 2 < HPW)
      def _():
        s_in(s + 2, b)

      do(s, b)
    return _

  lax.fori_loop(0, HPW // 2, step, None)
  pltpu.make_async_copy(ou0, o_hbm.at[pl.ds(off(HPW - 2), SLAB)],
                        sou0).wait()
  pltpu.make_async_copy(ou1, o_hbm.at[pl.ds(off(HPW - 1), SLAB)],
                        sou1).wait()


def _sc_body(idx_hbm, ch1_hbm, ch2_hbm, table_hbm, out_hbm,
             idxraw0, idxraw1, c1b0, c1b1, c2b0, c2b1,
             ib0, ib1, gb0, gb1, accv,
             sin0, sin1, sg0, sg1, nchunks):
  wid = lax.axis_index("s") * 2 + lax.axis_index("c")
  nt = (nchunks - wid + NW - 1) // NW
  lanes = _lanes()
  idxraws = (idxraw0, idxraw1)
  c1bs = (c1b0, c1b1)
  c2bs = (c2b0, c2b1)
  ibs = (ib0, ib1)
  gbs = (gb0, gb1)
  sins = (sin0, sin1)
  sgs = (sg0, sg1)

  def chunk_off(t):
    return (wid + NW * t) * LPC

  def s_in(t, b):
    off = chunk_off(t)
    pltpu.async_copy(idx_hbm.at[pl.ds(off, LPC)], idxraws[b], sins[b])
    pltpu.async_copy(ch1_hbm.at[pl.ds(off, LPC)], c1bs[b], sins[b])
    pltpu.async_copy(ch2_hbm.at[pl.ds(off, LPC)], c2bs[b], sins[b])

  def s_idx(t, b):
    off = chunk_off(t)
    pltpu.make_async_copy(idx_hbm.at[pl.ds(off, LPC)], idxraws[b],
                          sins[b]).wait()
    pltpu.make_async_copy(ch1_hbm.at[pl.ds(off, LPC)], c1bs[b],
                          sins[b]).wait()
    pltpu.make_async_copy(ch2_hbm.at[pl.ds(off, LPC)], c2bs[b],
                          sins[b]).wait()

    def build(u, _):
      offr = (u >> 3) * 256 + (u & 7) * 16
      r = idxraws[b][pl.ds(offr, 16)]
      c = idxraws[b][pl.ds(offr + 128, 16)]
      base = r * W + c
      btop = base - W
      qt = lax.shift_right_logical(btop, 3)
      qb = lax.shift_right_logical(base, 3)
      q = _splat(u >> 3)
      o = _splat((u & 7) << 4) + lanes
      plsc.store_scatter(ibs[b], [_splat(0), q, o], qt)
      plsc.store_scatter(ibs[b], [_splat(1), q, o], qt + 1)
      plsc.store_scatter(ibs[b], [_splat(2), q, o], qb)
      plsc.store_scatter(ibs[b], [_splat(3), q, o], qb + 1)
      plsc.store_scatter(ibs[b], [_splat(4), q, o], base & 7)
      return _

    lax.fori_loop(0, GROUPS_A, build, None)
    for k in range(4):
      for j in range(JROWS):
        pltpu.async_copy(table_hbm.at[ibs[b].at[k, j]], gbs[b].at[k, j],
                         sgs[b])

  def s_cmp(t, b):
    for k in range(4):
      for j in range(JROWS):
        pltpu.make_async_copy(table_hbm.at[ibs[b].at[k, j]],
                              gbs[b].at[k, j], sgs[b]).wait()

    def compute(v, acc):
      jv = _splat(v >> 4)
      rowp = _splat((v & 7) << 4) + lanes
      comp = _splat((v >> 3) & 1)
      a8 = plsc.load_gather(ibs[b], [_splat(4), jv, rowp])
      e01 = a8 * 2 + comp
      stra = (a8 == 7).astype(jnp.int32)
      e23 = (e01 + 2) & 15
      k0 = stra
      k2 = stra + 2
      p0 = plsc.load_gather(gbs[b], [_splat(0), jv, rowp, e01])
      p3 = plsc.load_gather(gbs[b], [k0, jv, rowp, e23])
      p1 = plsc.load_gather(gbs[b], [_splat(2), jv, rowp, e01])
      p2 = plsc.load_gather(gbs[b], [k2, jv, rowp, e23])
      c1v = c1bs[b][pl.ds(v * 16, 16)]
      c2v = c2bs[b][pl.ds(v * 16, 16)]
      tt = c2v - p1
      ca = 1.5 * (p1 - p2) + 0.5 * (p3 - p0)
      cb = p0 - 2.5 * p1 + 2.0 * p2 - 0.5 * p3
      cc = 0.5 * (p2 - p0)
      mapped = ((ca * tt + cb) * tt + cc) * tt + p1
      d = c1v - mapped
      return acc + d * d

    local = lax.fori_loop(0, VECS_B, compute,
                          jnp.zeros((L,), jnp.float32))
    accv[...] = accv[...] + local

  accv[...] = jnp.zeros((L,), jnp.float32)

  s_in(0, 0)

  @pl.when(nt > 1)
  def _():
    s_in(1, 1)

  s_idx(0, 0)

  def step(kk, _):
    for b in (0, 1):
      t = 2 * kk + b

      @pl.when(t + 1 < nt)
      def _():
        s_idx(t + 1, 1 - b)

      @pl.when(t < nt)
      def _():
        s_cmp(t, b)

      @pl.when(t + 2 < nt)
      def _():
        s_in(t + 2, b)
    return _

  kmax = (nchunks // NW + 2) // 2 + 1
  lax.fori_loop(0, kmax, step, None)

  pltpu.sync_copy(accv, out_hbm.at[wid])


_SC_PARAMS = pltpu.CompilerParams(
    use_tc_tiling_on_sc=False,
    needs_layout_passes=False,
)


def _make_relayout_call():
  mesh = plsc.VectorSubcoreMesh(core_axis_name="c", subcore_axis_name="s")
  return pl.kernel(
      _relayout_body,
      out_type=jax.ShapeDtypeStruct((H * W * 2,), jnp.float32),
      mesh=mesh,
      scratch_types=[
          pltpu.VMEM((SLAB,), jnp.float32),
          pltpu.VMEM((SLAB,), jnp.float32),
          pltpu.VMEM((SLAB,), jnp.float32),
          pltpu.VMEM((SLAB,), jnp.float32),
          pltpu.SemaphoreType.DMA,
          pltpu.SemaphoreType.DMA,
          pltpu.SemaphoreType.DMA,
          pltpu.SemaphoreType.DMA,
      ],
      compiler_params=_SC_PARAMS,
  )


def _make_sc_call(n_points):
  nchunks = n_points // T
  mesh = plsc.VectorSubcoreMesh(core_axis_name="c", subcore_axis_name="s")
  body = functools.partial(_sc_body, nchunks=nchunks)
  return pl.kernel(
      body,
      out_type=jax.ShapeDtypeStruct((NW, L), jnp.float32),
      mesh=mesh,
      scratch_types=[
          pltpu.VMEM((LPC,), jnp.int32),
          pltpu.VMEM((LPC,), jnp.int32),
          pltpu.VMEM((LPC,), jnp.float32),
          pltpu.VMEM((LPC,), jnp.float32),
          pltpu.VMEM((LPC,), jnp.float32),
          pltpu.VMEM((LPC,), jnp.float32),
          pltpu.VMEM((5, JROWS, 128), jnp.int32),
          pltpu.VMEM((5, JROWS, 128), jnp.int32),
          pltpu.VMEM((4, JROWS, 128, ROWW), jnp.float32),
          pltpu.VMEM((4, JROWS, 128, ROWW), jnp.float32),
          pltpu.VMEM((L,), jnp.float32),
          pltpu.SemaphoreType.DMA,
          pltpu.SemaphoreType.DMA,
          pltpu.SemaphoreType.DMA,
          pltpu.SemaphoreType.DMA,
      ],
      compiler_params=_SC_PARAMS,
  )


@jax.jit
def kernel(ch1, ch2, CP_locs, CP_idx):
  n = ch1.shape[0]
  nb = n // 128
  ch1f = ch1.reshape(nb, 128, 2).transpose(0, 2, 1).reshape(-1)
  ch2f = ch2.reshape(nb, 128, 2).transpose(0, 2, 1).reshape(-1)
  idxf = CP_idx.reshape(nb, 128, 2).transpose(0, 2, 1).reshape(-1)
  xf = CP_locs.reshape(H, W // 128, 128, 2).transpose(0, 1, 3, 2)
  xf = xf.reshape(-1)
  table = _make_relayout_call()(xf).reshape(VROWS, ROWW)
  partials = _make_sc_call(n)(idxf, ch1f, ch2f, table)
  return 0.5 * jnp.sum(partials) / n


# speedup vs baseline: 127.8678x; 1.0987x over previous
import functools

import jax
import jax.numpy as jnp
from jax import lax
from jax.experimental import pallas as pl
from jax.experimental.pallas import tpu as pltpu
from jax.experimental.pallas import tpu_sc as plsc

H = 2048
W = 2048
T = 640
LPC = 2 * T
GROUPS_A = T // 16
VECS_B = LPC // 16
JROWS = T // 128
NW = 32
L = 16
ROWW = 16
VROWS = H * W * 2 // ROWW
SLAB = 2 * W
HPW = H // NW


def _lanes():
  return lax.iota(jnp.int32, L)


def _splat(x):
  return jnp.broadcast_to(jnp.asarray(x, jnp.int32), (L,))


def _relayout_body(x_hbm, o_hbm, in0, in1, t00, t01, t40, t41,
                   sin0, sin1, sou0, sou1):
  wid = lax.axis_index("s") * 2 + lax.axis_index("c")
  lanes = _lanes()
  lh = (lanes >> 1) + (lanes & 1) * 128
  lhalf = lanes >> 1
  c128 = (lanes & 1) * 128
  ins = (in0, in1)
  t0s = (t00, t01)
  t4s = (t40, t41)
  sins = (sin0, sin1)
  sous = (sou0, sou1)
  half = H * SLAB

  def off(s):
    return (wid * HPW + s) * SLAB

  def s_in(s, b):
    o = off(s)
    pltpu.async_copy(x_hbm.at[pl.ds(o, SLAB)], ins[b].at[pl.ds(0, SLAB)],
                     sins[b])
    ho = lax.rem(o + SLAB, H * SLAB)
    pltpu.async_copy(x_hbm.at[pl.ds(ho, 256)],
                     ins[b].at[pl.ds(SLAB, 256)], sins[b])

  def s_in_wait(s, b):
    o = off(s)
    pltpu.make_async_copy(x_hbm.at[pl.ds(o, SLAB)],
                          ins[b].at[pl.ds(0, SLAB)], sins[b]).wait()
    ho = lax.rem(o + SLAB, H * SLAB)
    pltpu.make_async_copy(x_hbm.at[pl.ds(ho, 256)],
                          ins[b].at[pl.ds(SLAB, 256)], sins[b]).wait()

  def do(s, b):
    s_in_wait(s, b)

    @pl.when(s >= 2)
    def _():
      pltpu.make_async_copy(t0s[b], o_hbm.at[pl.ds(off(s - 2), SLAB)],
                            sous[b]).wait()
      pltpu.make_async_copy(t4s[b],
                            o_hbm.at[pl.ds(half + off(s - 2), SLAB)],
                            sous[b]).wait()

    def shuf(q, _):
      idx = _splat((q >> 4) * 256 + (q & 15) * 8) + lh
      t0s[b][pl.ds(q * 16, 16)] = plsc.load_gather(ins[b], [idx])
      cell = _splat((q & 15) * 8 + 4) + lhalf
      idx4 = (_splat((q >> 4) * 256) + c128 + (cell & 127)
              + lax.shift_left((cell >> 7), 8))
      t4s[b][pl.ds(q * 16, 16)] = plsc.load_gather(ins[b], [idx4])
      return _

    lax.fori_loop(0, SLAB // 16, shuf, None)
    pltpu.async_copy(t0s[b], o_hbm.at[pl.ds(off(s), SLAB)], sous[b])
    pltpu.async_copy(t4s[b], o_hbm.at[pl.ds(half + off(s), SLAB)],
                     sous[b])

  s_in(0, 0)
  s_in(1, 1)

  def step(kk, _):
    for b in (0, 1):
      s = 2 * kk + b
      do(s, b)

      @pl.when(s + 2 < HPW)
      def _():
        s_in(s + 2, b)
    return _

  lax.fori_loop(0, HPW // 2, step, None)
  for b in (0, 1):
    s = HPW - 2 + b
    pltpu.make_async_copy(t0s[b], o_hbm.at[pl.ds(off(s), SLAB)],
                          sous[b]).wait()
    pltpu.make_async_copy(t4s[b], o_hbm.at[pl.ds(half + off(s), SLAB)],
                          sous[b]).wait()


def _sc_body(idx_hbm, ch1_hbm, ch2_hbm, table_hbm, out_hbm,
             idxraw0, idxraw1, c1b0, c1b1, c2b0, c2b1,
             ib0, ib1, gb0, gb1, accv,
             sin0, sin1, sg0, sg1, nchunks):
  wid = lax.axis_index("s") * 2 + lax.axis_index("c")
  nt = (nchunks - wid + NW - 1) // NW
  lanes = _lanes()
  idxraws = (idxraw0, idxraw1)
  c1bs = (c1b0, c1b1)
  c2bs = (c2b0, c2b1)
  ibs = (ib0, ib1)
  gbs = (gb0, gb1)
  sins = (sin0, sin1)
  sgs = (sg0, sg1)

  def chunk_off(t):
    return (wid + NW * t) * LPC

  def s_in(t, b):
    off = chunk_off(t)
    pltpu.async_copy(idx_hbm.at[pl.ds(off, LPC)], idxraws[b], sins[b])
    pltpu.async_copy(ch1_hbm.at[pl.ds(off, LPC)], c1bs[b], sins[b])
    pltpu.async_copy(ch2_hbm.at[pl.ds(off, LPC)], c2bs[b], sins[b])

  def s_idx(t, b):
    off = chunk_off(t)
    pltpu.make_async_copy(idx_hbm.at[pl.ds(off, LPC)], idxraws[b],
                          sins[b]).wait()
    pltpu.make_async_copy(ch1_hbm.at[pl.ds(off, LPC)], c1bs[b],
                          sins[b]).wait()
    pltpu.make_async_copy(ch2_hbm.at[pl.ds(off, LPC)], c2bs[b],
                          sins[b]).wait()

    def build(u, _):
      offr = (u >> 3) * 256 + (u & 7) * 16
      r = idxraws[b][pl.ds(offr, 16)]
      c = idxraws[b][pl.ds(offr + 128, 16)]
      base = r * W + c
      a8 = base & 7
      stra = (a8 == 7).astype(jnp.int32)
      toff = stra * VROWS
      eb = jnp.where(a8 == 7, 6, a8 * 2)
      qt = lax.shift_right_logical(base - W, 3) + toff
      qb = lax.shift_right_logical(base, 3) + toff
      q = _splat(u >> 3)
      o = _splat((u & 7) << 4) + lanes
      plsc.store_scatter(ibs[b], [_splat(0), q, o], qt)
      plsc.store_scatter(ibs[b], [_splat(1), q, o], qb)
      plsc.store_scatter(ibs[b], [_splat(2), q, o], eb)
      return _

    lax.fori_loop(0, GROUPS_A, build, None)
    for k in range(2):
      for j in range(JROWS):
        pltpu.async_copy(table_hbm.at[ibs[b].at[k, j]], gbs[b].at[k, j],
                         sgs[b])

  def s_cmp(t, b):
    for k in range(2):
      for j in range(JROWS):
        pltpu.make_async_copy(table_hbm.at[ibs[b].at[k, j]],
                              gbs[b].at[k, j], sgs[b]).wait()

    def compute(v, acc):
      jv = _splat(v >> 4)
      rowp = _splat((v & 7) << 4) + lanes
      comp = _splat((v >> 3) & 1)
      eb = plsc.load_gather(ibs[b], [_splat(2), jv, rowp])
      e01 = eb + comp
      e23 = e01 + 2
      p0 = plsc.load_gather(gbs[b], [_splat(0), jv, rowp, e01])
      p3 = plsc.load_gather(gbs[b], [_splat(0), jv, rowp, e23])
      p1 = plsc.load_gather(gbs[b], [_splat(1), jv, rowp, e01])
      p2 = plsc.load_gather(gbs[b], [_splat(1), jv, rowp, e23])
      c1v = c1bs[b][pl.ds(v * 16, 16)]
      c2v = c2bs[b][pl.ds(v * 16, 16)]
      tt = c2v - p1
      ca = 1.5 * (p1 - p2) + 0.5 * (p3 - p0)
      cb = p0 - 2.5 * p1 + 2.0 * p2 - 0.5 * p3
      cc = 0.5 * (p2 - p0)
      mapped = ((ca * tt + cb) * tt + cc) * tt + p1
      d = c1v - mapped
      return acc + d * d

    local = lax.fori_loop(0, VECS_B, compute,
                          jnp.zeros((L,), jnp.float32))
    accv[...] = accv[...] + local

  accv[...] = jnp.zeros((L,), jnp.float32)

  s_in(0, 0)

  @pl.when(nt > 1)
  def _():
    s_in(1, 1)

  s_idx(0, 0)

  def step(kk, _):
    for b in (0, 1):
      t = 2 * kk + b

      @pl.when(t + 1 < nt)
      def _():
        s_idx(t + 1, 1 - b)

      @pl.when(t < nt)
      def _():
        s_cmp(t, b)

      @pl.when(t + 2 < nt)
      def _():
        s_in(t + 2, b)
    return _

  kmax = (nchunks // NW + 2) // 2 + 1
  lax.fori_loop(0, kmax, step, None)

  pltpu.sync_copy(accv, out_hbm.at[wid])


_SC_PARAMS = pltpu.CompilerParams(
    use_tc_tiling_on_sc=False,
    needs_layout_passes=False,
)


def _make_relayout_call():
  mesh = plsc.VectorSubcoreMesh(core_axis_name="c", subcore_axis_name="s")
  return pl.kernel(
      _relayout_body,
      out_type=jax.ShapeDtypeStruct((2 * H * W * 2,), jnp.float32),
      mesh=mesh,
      scratch_types=[
          pltpu.VMEM((SLAB + 256,), jnp.float32),
          pltpu.VMEM((SLAB + 256,), jnp.float32),
          pltpu.VMEM((SLAB,), jnp.float32),
          pltpu.VMEM((SLAB,), jnp.float32),
          pltpu.VMEM((SLAB,), jnp.float32),
          pltpu.VMEM((SLAB,), jnp.float32),
          pltpu.SemaphoreType.DMA,
          pltpu.SemaphoreType.DMA,
          pltpu.SemaphoreType.DMA,
          pltpu.SemaphoreType.DMA,
      ],
      compiler_params=_SC_PARAMS,
  )


def _make_sc_call(n_points):
  nchunks = n_points // T
  mesh = plsc.VectorSubcoreMesh(core_axis_name="c", subcore_axis_name="s")
  body = functools.partial(_sc_body, nchunks=nchunks)
  return pl.kernel(
      body,
      out_type=jax.ShapeDtypeStruct((NW, L), jnp.float32),
      mesh=mesh,
      scratch_types=[
          pltpu.VMEM((LPC,), jnp.int32),
          pltpu.VMEM((LPC,), jnp.int32),
          pltpu.VMEM((LPC,), jnp.float32),
          pltpu.VMEM((LPC,), jnp.float32),
          pltpu.VMEM((LPC,), jnp.float32),
          pltpu.VMEM((LPC,), jnp.float32),
          pltpu.VMEM((3, JROWS, 128), jnp.int32),
          pltpu.VMEM((3, JROWS, 128), jnp.int32),
          pltpu.VMEM((2, JROWS, 128, ROWW), jnp.float32),
          pltpu.VMEM((2, JROWS, 128, ROWW), jnp.float32),
          pltpu.VMEM((L,), jnp.float32),
          pltpu.SemaphoreType.DMA,
          pltpu.SemaphoreType.DMA,
          pltpu.SemaphoreType.DMA,
          pltpu.SemaphoreType.DMA,
      ],
      compiler_params=_SC_PARAMS,
  )


@jax.jit
def kernel(ch1, ch2, CP_locs, CP_idx):
  n = ch1.shape[0]
  nb = n // 128
  ch1f = ch1.reshape(nb, 128, 2).transpose(0, 2, 1).reshape(-1)
  ch2f = ch2.reshape(nb, 128, 2).transpose(0, 2, 1).reshape(-1)
  idxf = CP_idx.reshape(nb, 128, 2).transpose(0, 2, 1).reshape(-1)
  xf = CP_locs.reshape(H, W // 128, 128, 2).transpose(0, 1, 3, 2)
  xf = xf.reshape(-1)
  table = _make_relayout_call()(xf).reshape(2 * VROWS, ROWW)
  partials = _make_sc_call(n)(idxf, ch1f, ch2f, table)
  return 0.5 * jnp.sum(partials) / n


# speedup vs baseline: 143.5626x; 1.1227x over previous
import functools

import jax
import jax.numpy as jnp
from jax import lax
from jax.experimental import pallas as pl
from jax.experimental.pallas import tpu as pltpu
from jax.experimental.pallas import tpu_sc as plsc

H = 2048
W = 2048
T = 640
LPC = 2 * T
GROUPS_A = T // 16
VECS_B = LPC // 16
JROWS = T // 128
NW = 32
L = 16
ROWW = 16
VROWS = H * W * 2 // ROWW
SLAB = 2 * W
HPW = H // NW


def _lanes():
  return lax.iota(jnp.int32, L)


def _splat(x):
  return jnp.broadcast_to(jnp.asarray(x, jnp.int32), (L,))


def _relayout_body(x_hbm, o_hbm, in0, in1, t00, t01,
                   sin0, sin1, sou0, sou1):
  wid = lax.axis_index("s") * 2 + lax.axis_index("c")
  lanes = _lanes()
  lh = (lanes >> 1) + (lanes & 1) * 128
  ins = (in0, in1)
  t0s = (t00, t01)
  sins = (sin0, sin1)
  sous = (sou0, sou1)
  half = H * SLAB

  def off(s):
    return (wid * HPW + s) * SLAB

  def s_in(s, b):
    o = off(s)
    pltpu.async_copy(x_hbm.at[pl.ds(o, SLAB)], ins[b].at[pl.ds(0, SLAB)],
                     sins[b])
    ho = lax.rem(o + SLAB, H * SLAB)
    pltpu.async_copy(x_hbm.at[pl.ds(ho, 256)],
                     ins[b].at[pl.ds(SLAB, 256)], sins[b])

  def s_in_wait(s, b):
    o = off(s)
    pltpu.make_async_copy(x_hbm.at[pl.ds(o, SLAB)],
                          ins[b].at[pl.ds(0, SLAB)], sins[b]).wait()
    ho = lax.rem(o + SLAB, H * SLAB)
    pltpu.make_async_copy(x_hbm.at[pl.ds(ho, 256)],
                          ins[b].at[pl.ds(SLAB, 256)], sins[b]).wait()

  def do(s, b):
    s_in_wait(s, b)

    @pl.when(s >= 2)
    def _():
      pltpu.make_async_copy(t0s[b].at[pl.ds(0, SLAB)],
                            o_hbm.at[pl.ds(off(s - 2), SLAB)],
                            sous[b]).wait()
      pltpu.make_async_copy(t0s[b].at[pl.ds(8, SLAB)],
                            o_hbm.at[pl.ds(half + off(s - 2), SLAB)],
                            sous[b]).wait()

    def shuf(q, _):
      idx = _splat((q >> 4) * 256 + (q & 15) * 8) + lh
      t0s[b][pl.ds(q * 16, 16)] = plsc.load_gather(ins[b], [idx])
      return _

    lax.fori_loop(0, SLAB // 16, shuf, None)
    t0s[b][pl.ds(SLAB, 16)] = plsc.load_gather(ins[b], [_splat(SLAB) + lh])
    pltpu.async_copy(t0s[b].at[pl.ds(0, SLAB)],
                     o_hbm.at[pl.ds(off(s), SLAB)], sous[b])
    pltpu.async_copy(t0s[b].at[pl.ds(8, SLAB)],
                     o_hbm.at[pl.ds(half + off(s), SLAB)], sous[b])

  s_in(0, 0)
  s_in(1, 1)

  def step(kk, _):
    for b in (0, 1):
      s = 2 * kk + b
      do(s, b)

      @pl.when(s + 2 < HPW)
      def _():
        s_in(s + 2, b)
    return _

  lax.fori_loop(0, HPW // 2, step, None)
  for b in (0, 1):
    s = HPW - 2 + b
    pltpu.make_async_copy(t0s[b].at[pl.ds(0, SLAB)],
                          o_hbm.at[pl.ds(off(s), SLAB)], sous[b]).wait()
    pltpu.make_async_copy(t0s[b].at[pl.ds(8, SLAB)],
                          o_hbm.at[pl.ds(half + off(s), SLAB)],
                          sous[b]).wait()


def _sc_body(idx_hbm, ch1_hbm, ch2_hbm, table_hbm, out_hbm,
             idxraw0, idxraw1, c1b0, c1b1, c2b0, c2b1,
             ib0, ib1, gb0, gb1, accv,
             sin0, sin1, sg0, sg1, nchunks):
  wid = lax.axis_index("s") * 2 + lax.axis_index("c")
  nt = (nchunks - wid + NW - 1) // NW
  lanes = _lanes()
  idxraws = (idxraw0, idxraw1)
  c1bs = (c1b0, c1b1)
  c2bs = (c2b0, c2b1)
  ibs = (ib0, ib1)
  gbs = (gb0, gb1)
  sins = (sin0, sin1)
  sgs = (sg0, sg1)

  def chunk_off(t):
    return (wid + NW * t) * LPC

  def s_in(t, b):
    off = chunk_off(t)
    pltpu.async_copy(idx_hbm.at[pl.ds(off, LPC)], idxraws[b], sins[b])
    pltpu.async_copy(ch1_hbm.at[pl.ds(off, LPC)], c1bs[b], sins[b])
    pltpu.async_copy(ch2_hbm.at[pl.ds(off, LPC)], c2bs[b], sins[b])

  def s_idx(t, b):
    off = chunk_off(t)
    pltpu.make_async_copy(idx_hbm.at[pl.ds(off, LPC)], idxraws[b],
                          sins[b]).wait()
    pltpu.make_async_copy(ch1_hbm.at[pl.ds(off, LPC)], c1bs[b],
                          sins[b]).wait()
    pltpu.make_async_copy(ch2_hbm.at[pl.ds(off, LPC)], c2bs[b],
                          sins[b]).wait()

    def build(u, _):
      offr = (u >> 3) * 256 + (u & 7) * 16
      r = idxraws[b][pl.ds(offr, 16)]
      c = idxraws[b][pl.ds(offr + 128, 16)]
      base = r * W + c
      a8 = base & 7
      stra = (a8 == 7).astype(jnp.int32)
      toff = stra * VROWS
      eb = jnp.where(a8 == 7, 6, a8 * 2)
      qt = lax.shift_right_logical(base - W, 3) + toff
      qb = lax.shift_right_logical(base, 3) + toff
      q = _splat(u >> 3)
      o = _splat((u & 7) << 4) + lanes
      plsc.store_scatter(ibs[b], [_splat(0), q, o], qt)
      plsc.store_scatter(ibs[b], [_splat(1), q, o], qb)
      plsc.store_scatter(ibs[b], [_splat(2), q, o], eb)
      return _

    lax.fori_loop(0, GROUPS_A, build, None)
    for k in range(2):
      for j in range(JROWS):
        pltpu.async_copy(table_hbm.at[ibs[b].at[k, j]], gbs[b].at[k, j],
                         sgs[b])

  def s_cmp(t, b):
    for k in range(2):
      for j in range(JROWS):
        pltpu.make_async_copy(table_hbm.at[ibs[b].at[k, j]],
                              gbs[b].at[k, j], sgs[b]).wait()

    def compute(v, acc):
      jv = _splat(v >> 4)
      rowp = _splat((v & 7) << 4) + lanes
      comp = _splat((v >> 3) & 1)
      eb = plsc.load_gather(ibs[b], [_splat(2), jv, rowp])
      e01 = eb + comp
      e23 = e01 + 2
      p0 = plsc.load_gather(gbs[b], [_splat(0), jv, rowp, e01])
      p3 = plsc.load_gather(gbs[b], [_splat(0), jv, rowp, e23])
      p1 = plsc.load_gather(gbs[b], [_splat(1), jv, rowp, e01])
      p2 = plsc.load_gather(gbs[b], [_splat(1), jv, rowp, e23])
      c1v = c1bs[b][pl.ds(v * 16, 16)]
      c2v = c2bs[b][pl.ds(v * 16, 16)]
      tt = c2v - p1
      ca = 1.5 * (p1 - p2) + 0.5 * (p3 - p0)
      cb = p0 - 2.5 * p1 + 2.0 * p2 - 0.5 * p3
      cc = 0.5 * (p2 - p0)
      mapped = ((ca * tt + cb) * tt + cc) * tt + p1
      d = c1v - mapped
      return acc + d * d

    local = lax.fori_loop(0, VECS_B, compute,
                          jnp.zeros((L,), jnp.float32))
    accv[...] = accv[...] + local

  accv[...] = jnp.zeros((L,), jnp.float32)

  s_in(0, 0)

  @pl.when(nt > 1)
  def _():
    s_in(1, 1)

  s_idx(0, 0)

  def step(kk, _):
    for b in (0, 1):
      t = 2 * kk + b

      @pl.when(t + 1 < nt)
      def _():
        s_idx(t + 1, 1 - b)

      @pl.when(t < nt)
      def _():
        s_cmp(t, b)

      @pl.when(t + 2 < nt)
      def _():
        s_in(t + 2, b)
    return _

  kmax = (nchunks // NW + 2) // 2 + 1
  lax.fori_loop(0, kmax, step, None)

  pltpu.sync_copy(accv, out_hbm.at[wid])


_SC_PARAMS = pltpu.CompilerParams(
    use_tc_tiling_on_sc=False,
    needs_layout_passes=False,
)


def _make_relayout_call():
  mesh = plsc.VectorSubcoreMesh(core_axis_name="c", subcore_axis_name="s")
  return pl.kernel(
      _relayout_body,
      out_type=jax.ShapeDtypeStruct((2 * H * W * 2,), jnp.float32),
      mesh=mesh,
      scratch_types=[
          pltpu.VMEM((SLAB + 256,), jnp.float32),
          pltpu.VMEM((SLAB + 256,), jnp.float32),
          pltpu.VMEM((SLAB + 16,), jnp.float32),
          pltpu.VMEM((SLAB + 16,), jnp.float32),
          pltpu.SemaphoreType.DMA,
          pltpu.SemaphoreType.DMA,
          pltpu.SemaphoreType.DMA,
          pltpu.SemaphoreType.DMA,
      ],
      compiler_params=_SC_PARAMS,
  )


def _make_sc_call(n_points):
  nchunks = n_points // T
  mesh = plsc.VectorSubcoreMesh(core_axis_name="c", subcore_axis_name="s")
  body = functools.partial(_sc_body, nchunks=nchunks)
  return pl.kernel(
      body,
      out_type=jax.ShapeDtypeStruct((NW, L), jnp.float32),
      mesh=mesh,
      scratch_types=[
          pltpu.VMEM((LPC,), jnp.int32),
          pltpu.VMEM((LPC,), jnp.int32),
          pltpu.VMEM((LPC,), jnp.float32),
          pltpu.VMEM((LPC,), jnp.float32),
          pltpu.VMEM((LPC,), jnp.float32),
          pltpu.VMEM((LPC,), jnp.float32),
          pltpu.VMEM((3, JROWS, 128), jnp.int32),
          pltpu.VMEM((3, JROWS, 128), jnp.int32),
          pltpu.VMEM((2, JROWS, 128, ROWW), jnp.float32),
          pltpu.VMEM((2, JROWS, 128, ROWW), jnp.float32),
          pltpu.VMEM((L,), jnp.float32),
          pltpu.SemaphoreType.DMA,
          pltpu.SemaphoreType.DMA,
          pltpu.SemaphoreType.DMA,
          pltpu.SemaphoreType.DMA,
      ],
      compiler_params=_SC_PARAMS,
  )


@jax.jit
def kernel(ch1, ch2, CP_locs, CP_idx):
  n = ch1.shape[0]
  nb = n // 128
  ch1f = ch1.reshape(nb, 128, 2).transpose(0, 2, 1).reshape(-1)
  ch2f = ch2.reshape(nb, 128, 2).transpose(0, 2, 1).reshape(-1)
  idxf = CP_idx.reshape(nb, 128, 2).transpose(0, 2, 1).reshape(-1)
  xf = CP_locs.reshape(H, W // 128, 128, 2).transpose(0, 1, 3, 2)
  xf = xf.reshape(-1)
  table = _make_relayout_call()(xf).reshape(2 * VROWS, ROWW)
  partials = _make_sc_call(n)(idxf, ch1f, ch2f, table)
  return 0.5 * jnp.sum(partials) / n
